# baseline (speedup 1.0000x reference)
import functools

import jax
import jax.numpy as jnp
from jax import lax
from jax.experimental import pallas as pl
from jax.experimental.pallas import tpu as pltpu
from jax.experimental.pallas import tpu_sc as plsc

WIN = 5
COUT = 200
CD = 50
CH = 12
WD = 300
CV = 128


def _sc_gather_rows(table, idx):
    T = idx.shape[0]
    NWORK = 32
    per_w = T // NWORK
    K = 16
    LAG = 1
    nch = per_w // K
    mesh = plsc.VectorSubcoreMesh(core_axis_name="c", subcore_axis_name="s")

    @functools.partial(
        pl.kernel,
        mesh=mesh,
        out_type=jax.ShapeDtypeStruct((T, WD), jnp.float32),
        scratch_types=[
            pltpu.VMEM((per_w,), jnp.int32),
            pltpu.SemaphoreType.DMA,
        ],
    )
    def gk(table_hbm, idx_hbm, out_hbm, idx_s, sem):
        wid = lax.axis_index("s") * 2 + lax.axis_index("c")
        base = wid * per_w
        pltpu.sync_copy(idx_hbm.at[pl.ds(base, per_w)], idx_s)

        def drain_one_chunk():
            pltpu.make_async_copy(
                table_hbm.at[pl.ds(0, K)], out_hbm.at[pl.ds(base, K)], sem).wait()

        def chunk(jc, carry):
            r0 = jc * K
            for b0 in range(0, K, 16):
                iv = idx_s[pl.ds(r0 + b0, 16)]
                for b in range(16):
                    pltpu.async_copy(
                        table_hbm.at[iv[b]], out_hbm.at[base + r0 + b0 + b], sem)
            @pl.when(jc >= LAG)
            def _():
                drain_one_chunk()
            return carry

        lax.fori_loop(0, nch, chunk, 0)
        for _ in range(min(LAG, nch)):
            drain_one_chunk()

    return gk(table, idx)


def _char_word_block(w_ref, ch_ref, tab_ref, wf_ref, b_ref, out_ref):
    NWB = ch_ref.shape[0]
    cb = ch_ref[...]
    tab = tab_ref[...]
    dn = (((1,), (0,)), ((), ()))
    emb = []
    for t in range(CH):
        oh = (cb[:, t : t + 1]
              == lax.broadcasted_iota(jnp.int32, (NWB, CV), 1)).astype(jnp.float32)
        emb.append(lax.dot_general(oh, tab, dn, preferred_element_type=jnp.float32))
    zero = jnp.zeros((NWB, CD), jnp.float32)
    wf = wf_ref[...]
    m = None
    for t in range(CH):
        parts = []
        for s in range(-2, 3):
            tt = t + s
            parts.append(emb[tt] if 0 <= tt < CH else zero)
        x = jnp.concatenate(parts, axis=1)
        y = lax.dot_general(x, wf, dn, preferred_element_type=jnp.float32)
        m = y if m is None else jnp.maximum(m, y)
    feat = m + b_ref[...]
    out_ref[...] = jnp.concatenate([w_ref[...], feat], axis=1)


def _char_concat(word_rows, chars_flat, tab, wflat, bias2, row_off):
    n_tok = chars_flat.shape[0]
    NWB = 256
    grid = (n_tok // NWB,)
    off_blk = row_off // NWB
    return pl.pallas_call(
        _char_word_block,
        grid=grid,
        in_specs=[
            pl.BlockSpec((NWB, WD), lambda i: (off_blk + i, 0)),
            pl.BlockSpec((NWB, CH), lambda i: (i, 0)),
            pl.BlockSpec((CV, CD), lambda i: (0, 0)),
            pl.BlockSpec((WIN * CD, COUT), lambda i: (0, 0)),
            pl.BlockSpec((1, COUT), lambda i: (0, 0)),
        ],
        out_specs=pl.BlockSpec((NWB, WD + COUT), lambda i: (i, 0)),
        out_shape=jax.ShapeDtypeStruct((n_tok, WD + COUT), jnp.float32),
    )(word_rows, chars_flat, tab, wflat, bias2)


def _onehot_matmul(iref, tref, vocab):
    oh = (iref[...]
          == lax.broadcasted_iota(jnp.int32, (iref.shape[0], vocab), 1)
          ).astype(jnp.float32)
    return lax.dot_general(oh, tref[...], (((1,), (0,)), ((), ())),
                           preferred_element_type=jnp.float32)


def _p_small_block(pos_i, ner_i, r1_i, r2_i, pos_t, ner_t, rel_t, o1, o2, o3, o4):
    o1[...] = _onehot_matmul(pos_i, pos_t, 50)
    o2[...] = _onehot_matmul(ner_i, ner_t, 20)
    o3[...] = _onehot_matmul(r1_i, rel_t, 40)
    o4[...] = _onehot_matmul(r2_i, rel_t, 40)


def _p_small(pPos, pNer, pQRel, pCRel, pos_table, ner_table, rel_table):
    n_tok = pPos.shape[0]
    NT = 512
    grid = (n_tok // NT,)
    idx_spec = pl.BlockSpec((NT, 1), lambda i: (i, 0))
    full = lambda shp: pl.BlockSpec(shp, lambda i: (0, 0))
    return pl.pallas_call(
        _p_small_block,
        grid=grid,
        in_specs=[idx_spec, idx_spec, idx_spec, idx_spec,
                  full(pos_table.shape), full(ner_table.shape), full(rel_table.shape)],
        out_specs=[pl.BlockSpec((NT, 12), lambda i: (i, 0)),
                   pl.BlockSpec((NT, 8), lambda i: (i, 0)),
                   pl.BlockSpec((NT, 10), lambda i: (i, 0)),
                   pl.BlockSpec((NT, 10), lambda i: (i, 0))],
        out_shape=[jax.ShapeDtypeStruct((n_tok, 12), jnp.float32),
                   jax.ShapeDtypeStruct((n_tok, 8), jnp.float32),
                   jax.ShapeDtypeStruct((n_tok, 10), jnp.float32),
                   jax.ShapeDtypeStruct((n_tok, 10), jnp.float32)],
    )(pPos, pNer, pQRel, pCRel, pos_table, ner_table, rel_table)


def _q_small_block(pos_i, pos_t, o1):
    o1[...] = _onehot_matmul(pos_i, pos_t, 50)


def _q_small(qPos, pos_table):
    n_tok = qPos.shape[0]
    NT = 512
    grid = (n_tok // NT,)
    return pl.pallas_call(
        _q_small_block,
        grid=grid,
        in_specs=[pl.BlockSpec((NT, 1), lambda i: (i, 0)),
                  pl.BlockSpec(pos_table.shape, lambda i: (0, 0))],
        out_specs=pl.BlockSpec((NT, 12), lambda i: (i, 0)),
        out_shape=jax.ShapeDtypeStruct((n_tok, 12), jnp.float32),
    )(qPos, pos_table)


def kernel(p, q, c, pPos, pNer, qPos, pQRel, pCRel, pChars, qChars, cChars,
           word_table, pos_table, ner_table, rel_table, char_table, conv_w, conv_b):
    B, PL = p.shape
    QL = q.shape[1]
    CL = c.shape[1]
    i32 = jnp.int32

    idx_all = jnp.concatenate(
        [p.reshape(-1), q.reshape(-1), c.reshape(-1)]).astype(i32)
    rows = _sc_gather_rows(word_table, idx_all)

    wflat = conv_w.transpose(2, 1, 0).reshape(WIN * CD, COUT)
    bias2 = conv_b.reshape(1, COUT)

    pEmb = _char_concat(rows, pChars.reshape(B * PL, CH).astype(i32),
                        char_table, wflat, bias2, 0).reshape(B, PL, WD + COUT)
    qEmb = _char_concat(rows, qChars.reshape(B * QL, CH).astype(i32),
                        char_table, wflat, bias2, B * PL).reshape(B, QL, WD + COUT)
    cEmb = _char_concat(rows, cChars.reshape(B * CL, CH).astype(i32),
                        char_table, wflat, bias2, B * (PL + QL)).reshape(B, CL, WD + COUT)

    o1, o2, o3, o4 = _p_small(pPos.reshape(B * PL, 1).astype(i32),
                              pNer.reshape(B * PL, 1).astype(i32),
                              pQRel.reshape(B * PL, 1).astype(i32),
                              pCRel.reshape(B * PL, 1).astype(i32),
                              pos_table, ner_table, rel_table)
    qPosEmb = _q_small(qPos.reshape(B * QL, 1).astype(i32),
                       pos_table).reshape(B, QL, 12)

    return (pEmb, qEmb, cEmb,
            o1.reshape(B, PL, 12), o2.reshape(B, PL, 8), qPosEmb,
            o3.reshape(B, PL, 10), o4.reshape(B, PL, 10))


# speedup vs baseline: 1.4594x; 1.4594x over previous
import functools

import jax
import jax.numpy as jnp
from jax import lax
from jax.experimental import pallas as pl
from jax.experimental.pallas import tpu as pltpu
from jax.experimental.pallas import tpu_sc as plsc

WIN = 5
COUT = 200
CD = 50
CH = 12
WD = 300
CV = 128


WDP = 384


def _sc_gather_rows(table384, idx):
    T = idx.shape[0]
    NWORK = 32
    per_w = T // NWORK
    CHUNK = 128
    nch = per_w // CHUNK
    idx3d = idx.reshape(NWORK, per_w // CHUNK, CHUNK)
    mesh = plsc.VectorSubcoreMesh(core_axis_name="c", subcore_axis_name="s")

    @functools.partial(
        pl.kernel,
        mesh=mesh,
        out_type=jax.ShapeDtypeStruct((T, WDP), jnp.float32),
        scratch_types=[
            pltpu.VMEM((nch, CHUNK), jnp.int32),
            pltpu.VMEM((2, CHUNK, WDP), jnp.float32),
            pltpu.SemaphoreType.DMA,
            pltpu.SemaphoreType.DMA,
        ],
    )
    def gk(table_hbm, idx_hbm, out_hbm, idx_v, rows_v, sem_g, sem_o):
        wid = lax.axis_index("s") * 2 + lax.axis_index("c")
        base = wid * per_w
        pltpu.sync_copy(idx_hbm.at[wid], idx_v)
        gathers = []
        outs = [None, None]
        for j in range(nch):
            b = j % 2
            if outs[b] is not None:
                outs[b].wait()
                outs[b] = None
            gathers.append(
                pltpu.async_copy(table_hbm.at[idx_v.at[j]], rows_v.at[b], sem_g))
            gathers[j].wait()
            outs[b] = pltpu.async_copy(
                rows_v.at[b], out_hbm.at[pl.ds(base + j * CHUNK, CHUNK)], sem_o)
        for o in outs:
            if o is not None:
                o.wait()

    return gk(table384, idx3d)


def _char_word_block(w_ref, ch_ref, tab_ref, wf_ref, b_ref, out_ref):
    NWB = ch_ref.shape[0]
    cb = ch_ref[...]
    tab = tab_ref[...]
    dn = (((1,), (0,)), ((), ()))
    emb = []
    for t in range(CH):
        oh = (cb[:, t : t + 1]
              == lax.broadcasted_iota(jnp.int32, (NWB, CV), 1)).astype(jnp.float32)
        emb.append(lax.dot_general(oh, tab, dn, preferred_element_type=jnp.float32))
    zero = jnp.zeros((NWB, CD), jnp.float32)
    wf = wf_ref[...]
    m = None
    for t in range(CH):
        parts = []
        for s in range(-2, 3):
            tt = t + s
            parts.append(emb[tt] if 0 <= tt < CH else zero)
        x = jnp.concatenate(parts, axis=1)
        y = lax.dot_general(x, wf, dn, preferred_element_type=jnp.float32)
        m = y if m is None else jnp.maximum(m, y)
    feat = m + b_ref[...]
    out_ref[...] = jnp.concatenate([w_ref[:, :WD], feat], axis=1)


def _char_concat(word_rows, chars_flat, tab, wflat, bias2, row_off):
    n_tok = chars_flat.shape[0]
    NWB = 256
    grid = (n_tok // NWB,)
    off_blk = row_off // NWB
    return pl.pallas_call(
        _char_word_block,
        grid=grid,
        in_specs=[
            pl.BlockSpec((NWB, WDP), lambda i: (off_blk + i, 0)),
            pl.BlockSpec((NWB, CH), lambda i: (i, 0)),
            pl.BlockSpec((CV, CD), lambda i: (0, 0)),
            pl.BlockSpec((WIN * CD, COUT), lambda i: (0, 0)),
            pl.BlockSpec((1, COUT), lambda i: (0, 0)),
        ],
        out_specs=pl.BlockSpec((NWB, WD + COUT), lambda i: (i, 0)),
        out_shape=jax.ShapeDtypeStruct((n_tok, WD + COUT), jnp.float32),
    )(word_rows, chars_flat, tab, wflat, bias2)


def _onehot_matmul(iref, tref, vocab):
    oh = (iref[...]
          == lax.broadcasted_iota(jnp.int32, (iref.shape[0], vocab), 1)
          ).astype(jnp.float32)
    return lax.dot_general(oh, tref[...], (((1,), (0,)), ((), ())),
                           preferred_element_type=jnp.float32)


def _p_small_block(pos_i, ner_i, r1_i, r2_i, pos_t, ner_t, rel_t, o1, o2, o3, o4):
    o1[...] = _onehot_matmul(pos_i, pos_t, 50)
    o2[...] = _onehot_matmul(ner_i, ner_t, 20)
    o3[...] = _onehot_matmul(r1_i, rel_t, 40)
    o4[...] = _onehot_matmul(r2_i, rel_t, 40)


def _p_small(pPos, pNer, pQRel, pCRel, pos_table, ner_table, rel_table):
    n_tok = pPos.shape[0]
    NT = 512
    grid = (n_tok // NT,)
    idx_spec = pl.BlockSpec((NT, 1), lambda i: (i, 0))
    full = lambda shp: pl.BlockSpec(shp, lambda i: (0, 0))
    return pl.pallas_call(
        _p_small_block,
        grid=grid,
        in_specs=[idx_spec, idx_spec, idx_spec, idx_spec,
                  full(pos_table.shape), full(ner_table.shape), full(rel_table.shape)],
        out_specs=[pl.BlockSpec((NT, 12), lambda i: (i, 0)),
                   pl.BlockSpec((NT, 8), lambda i: (i, 0)),
                   pl.BlockSpec((NT, 10), lambda i: (i, 0)),
                   pl.BlockSpec((NT, 10), lambda i: (i, 0))],
        out_shape=[jax.ShapeDtypeStruct((n_tok, 12), jnp.float32),
                   jax.ShapeDtypeStruct((n_tok, 8), jnp.float32),
                   jax.ShapeDtypeStruct((n_tok, 10), jnp.float32),
                   jax.ShapeDtypeStruct((n_tok, 10), jnp.float32)],
    )(pPos, pNer, pQRel, pCRel, pos_table, ner_table, rel_table)


def _q_small_block(pos_i, pos_t, o1):
    o1[...] = _onehot_matmul(pos_i, pos_t, 50)


def _q_small(qPos, pos_table):
    n_tok = qPos.shape[0]
    NT = 512
    grid = (n_tok // NT,)
    return pl.pallas_call(
        _q_small_block,
        grid=grid,
        in_specs=[pl.BlockSpec((NT, 1), lambda i: (i, 0)),
                  pl.BlockSpec(pos_table.shape, lambda i: (0, 0))],
        out_specs=pl.BlockSpec((NT, 12), lambda i: (i, 0)),
        out_shape=jax.ShapeDtypeStruct((n_tok, 12), jnp.float32),
    )(qPos, pos_table)


def kernel(p, q, c, pPos, pNer, qPos, pQRel, pCRel, pChars, qChars, cChars,
           word_table, pos_table, ner_table, rel_table, char_table, conv_w, conv_b):
    B, PL = p.shape
    QL = q.shape[1]
    CL = c.shape[1]
    i32 = jnp.int32

    idx_all = jnp.concatenate(
        [p.reshape(-1), q.reshape(-1), c.reshape(-1)]).astype(i32)
    table384 = jnp.pad(word_table, ((0, 0), (0, WDP - WD)))
    rows = _sc_gather_rows(table384, idx_all)

    wflat = conv_w.transpose(2, 1, 0).reshape(WIN * CD, COUT)
    bias2 = conv_b.reshape(1, COUT)

    pEmb = _char_concat(rows, pChars.reshape(B * PL, CH).astype(i32),
                        char_table, wflat, bias2, 0).reshape(B, PL, WD + COUT)
    qEmb = _char_concat(rows, qChars.reshape(B * QL, CH).astype(i32),
                        char_table, wflat, bias2, B * PL).reshape(B, QL, WD + COUT)
    cEmb = _char_concat(rows, cChars.reshape(B * CL, CH).astype(i32),
                        char_table, wflat, bias2, B * (PL + QL)).reshape(B, CL, WD + COUT)

    o1, o2, o3, o4 = _p_small(pPos.reshape(B * PL, 1).astype(i32),
                              pNer.reshape(B * PL, 1).astype(i32),
                              pQRel.reshape(B * PL, 1).astype(i32),
                              pCRel.reshape(B * PL, 1).astype(i32),
                              pos_table, ner_table, rel_table)
    qPosEmb = _q_small(qPos.reshape(B * QL, 1).astype(i32),
                       pos_table).reshape(B, QL, 12)

    return (pEmb, qEmb, cEmb,
            o1.reshape(B, PL, 12), o2.reshape(B, PL, 8), qPosEmb,
            o3.reshape(B, PL, 10), o4.reshape(B, PL, 10))


# speedup vs baseline: 2.4185x; 1.6571x over previous
import functools

import jax
import jax.numpy as jnp
from jax import lax
from jax.experimental import pallas as pl
from jax.experimental.pallas import tpu as pltpu
from jax.experimental.pallas import tpu_sc as plsc

WIN = 5
COUT = 200
CD = 50
CH = 12
WD = 300
CV = 128


LW = 128


def _sc_gather_rows(table, tail128, idx):
    T = idx.shape[0]
    NWORK = 32
    per_w = T // NWORK
    CHUNK = 128
    nch = per_w // CHUNK
    idx3d = idx.reshape(NWORK, nch, CHUNK)
    mesh = plsc.VectorSubcoreMesh(core_axis_name="c", subcore_axis_name="s")
    out_t = jax.ShapeDtypeStruct((T, LW), jnp.float32)

    @functools.partial(
        pl.kernel,
        mesh=mesh,
        out_type=(out_t, out_t, out_t),
        scratch_types=[
            pltpu.VMEM((nch, CHUNK), jnp.int32),
            pltpu.VMEM((2, 3, CHUNK, LW), jnp.float32),
            pltpu.SemaphoreType.DMA,
            pltpu.SemaphoreType.DMA,
        ],
    )
    def gk(table_hbm, tail_hbm, idx_hbm, o0, o1, o2, idx_v, rows_v, sem_g, sem_o):
        wid = lax.axis_index("s") * 2 + lax.axis_index("c")
        base = wid * per_w
        pltpu.sync_copy(idx_hbm.at[wid], idx_v)
        srcs = [table_hbm.at[:, pl.ds(0, LW)],
                table_hbm.at[:, pl.ds(LW, LW)],
                tail_hbm]
        dsts = [o0, o1, o2]
        outs = [None, None]
        for j in range(nch):
            b = j % 2
            if outs[b] is not None:
                for o in outs[b]:
                    o.wait()
                outs[b] = None
            gs = [pltpu.async_copy(srcs[k].at[idx_v.at[j]], rows_v.at[b, k], sem_g)
                  for k in range(3)]
            for g in gs:
                g.wait()
            outs[b] = [
                pltpu.async_copy(
                    rows_v.at[b, k],
                    dsts[k].at[pl.ds(base + j * CHUNK, CHUNK)], sem_o)
                for k in range(3)]
        for ob in outs:
            if ob is not None:
                for o in ob:
                    o.wait()

    return gk(table, tail128, idx3d)


def _char_word_block(w0_ref, w1_ref, w2_ref, ch_ref, tab_ref, wf_ref, b_ref, out_ref):
    NWB = ch_ref.shape[0]
    cb = ch_ref[...]
    tab = tab_ref[...]
    dn = (((1,), (0,)), ((), ()))
    emb = []
    for t in range(CH):
        oh = (cb[:, t : t + 1]
              == lax.broadcasted_iota(jnp.int32, (NWB, CV), 1)).astype(jnp.float32)
        emb.append(lax.dot_general(oh, tab, dn, preferred_element_type=jnp.float32))
    zero = jnp.zeros((NWB, CD), jnp.float32)
    wf = wf_ref[...]
    m = None
    for t in range(CH):
        parts = []
        for s in range(-2, 3):
            tt = t + s
            parts.append(emb[tt] if 0 <= tt < CH else zero)
        x = jnp.concatenate(parts, axis=1)
        y = lax.dot_general(x, wf, dn, preferred_element_type=jnp.float32)
        m = y if m is None else jnp.maximum(m, y)
    feat = m + b_ref[...]
    out_ref[...] = jnp.concatenate(
        [w0_ref[...], w1_ref[...], w2_ref[:, : WD - 2 * LW], feat], axis=1)


def _char_concat(w0, w1, w2, chars_flat, tab, wflat, bias2, row_off):
    n_tok = chars_flat.shape[0]
    NWB = 256
    grid = (n_tok // NWB,)
    off_blk = row_off // NWB
    wspec = pl.BlockSpec((NWB, LW), lambda i: (off_blk + i, 0))
    return pl.pallas_call(
        _char_word_block,
        grid=grid,
        in_specs=[
            wspec, wspec, wspec,
            pl.BlockSpec((NWB, CH), lambda i: (i, 0)),
            pl.BlockSpec((CV, CD), lambda i: (0, 0)),
            pl.BlockSpec((WIN * CD, COUT), lambda i: (0, 0)),
            pl.BlockSpec((1, COUT), lambda i: (0, 0)),
        ],
        out_specs=pl.BlockSpec((NWB, WD + COUT), lambda i: (i, 0)),
        out_shape=jax.ShapeDtypeStruct((n_tok, WD + COUT), jnp.float32),
    )(w0, w1, w2, chars_flat, tab, wflat, bias2)


def _onehot_matmul(iref, tref, vocab):
    oh = (iref[...]
          == lax.broadcasted_iota(jnp.int32, (iref.shape[0], vocab), 1)
          ).astype(jnp.float32)
    return lax.dot_general(oh, tref[...], (((1,), (0,)), ((), ())),
                           preferred_element_type=jnp.float32)


def _p_small_block(pos_i, ner_i, r1_i, r2_i, pos_t, ner_t, rel_t, o1, o2, o3, o4):
    o1[...] = _onehot_matmul(pos_i, pos_t, 50)
    o2[...] = _onehot_matmul(ner_i, ner_t, 20)
    o3[...] = _onehot_matmul(r1_i, rel_t, 40)
    o4[...] = _onehot_matmul(r2_i, rel_t, 40)


def _p_small(pPos, pNer, pQRel, pCRel, pos_table, ner_table, rel_table):
    n_tok = pPos.shape[0]
    NT = 512
    grid = (n_tok // NT,)
    idx_spec = pl.BlockSpec((NT, 1), lambda i: (i, 0))
    full = lambda shp: pl.BlockSpec(shp, lambda i: (0, 0))
    return pl.pallas_call(
        _p_small_block,
        grid=grid,
        in_specs=[idx_spec, idx_spec, idx_spec, idx_spec,
                  full(pos_table.shape), full(ner_table.shape), full(rel_table.shape)],
        out_specs=[pl.BlockSpec((NT, 12), lambda i: (i, 0)),
                   pl.BlockSpec((NT, 8), lambda i: (i, 0)),
                   pl.BlockSpec((NT, 10), lambda i: (i, 0)),
                   pl.BlockSpec((NT, 10), lambda i: (i, 0))],
        out_shape=[jax.ShapeDtypeStruct((n_tok, 12), jnp.float32),
                   jax.ShapeDtypeStruct((n_tok, 8), jnp.float32),
                   jax.ShapeDtypeStruct((n_tok, 10), jnp.float32),
                   jax.ShapeDtypeStruct((n_tok, 10), jnp.float32)],
    )(pPos, pNer, pQRel, pCRel, pos_table, ner_table, rel_table)


def _q_small_block(pos_i, pos_t, o1):
    o1[...] = _onehot_matmul(pos_i, pos_t, 50)


def _q_small(qPos, pos_table):
    n_tok = qPos.shape[0]
    NT = 512
    grid = (n_tok // NT,)
    return pl.pallas_call(
        _q_small_block,
        grid=grid,
        in_specs=[pl.BlockSpec((NT, 1), lambda i: (i, 0)),
                  pl.BlockSpec(pos_table.shape, lambda i: (0, 0))],
        out_specs=pl.BlockSpec((NT, 12), lambda i: (i, 0)),
        out_shape=jax.ShapeDtypeStruct((n_tok, 12), jnp.float32),
    )(qPos, pos_table)


def kernel(p, q, c, pPos, pNer, qPos, pQRel, pCRel, pChars, qChars, cChars,
           word_table, pos_table, ner_table, rel_table, char_table, conv_w, conv_b):
    B, PL = p.shape
    QL = q.shape[1]
    CL = c.shape[1]
    i32 = jnp.int32

    idx_all = jnp.concatenate(
        [p.reshape(-1), q.reshape(-1), c.reshape(-1)]).astype(i32)
    tail128 = jnp.pad(word_table[:, 2 * LW :], ((0, 0), (0, 3 * LW - WD)))
    w0, w1, w2 = _sc_gather_rows(word_table, tail128, idx_all)

    wflat = conv_w.transpose(2, 1, 0).reshape(WIN * CD, COUT)
    bias2 = conv_b.reshape(1, COUT)

    pEmb = _char_concat(w0, w1, w2, pChars.reshape(B * PL, CH).astype(i32),
                        char_table, wflat, bias2, 0).reshape(B, PL, WD + COUT)
    qEmb = _char_concat(w0, w1, w2, qChars.reshape(B * QL, CH).astype(i32),
                        char_table, wflat, bias2, B * PL).reshape(B, QL, WD + COUT)
    cEmb = _char_concat(w0, w1, w2, cChars.reshape(B * CL, CH).astype(i32),
                        char_table, wflat, bias2, B * (PL + QL)).reshape(B, CL, WD + COUT)

    o1, o2, o3, o4 = _p_small(pPos.reshape(B * PL, 1).astype(i32),
                              pNer.reshape(B * PL, 1).astype(i32),
                              pQRel.reshape(B * PL, 1).astype(i32),
                              pCRel.reshape(B * PL, 1).astype(i32),
                              pos_table, ner_table, rel_table)
    qPosEmb = _q_small(qPos.reshape(B * QL, 1).astype(i32),
                       pos_table).reshape(B, QL, 12)

    return (pEmb, qEmb, cEmb,
            o1.reshape(B, PL, 12), o2.reshape(B, PL, 8), qPosEmb,
            o3.reshape(B, PL, 10), o4.reshape(B, PL, 10))


# speedup vs baseline: 2.4257x; 1.0030x over previous
import functools

import jax
import jax.numpy as jnp
from jax import lax
from jax.experimental import pallas as pl
from jax.experimental.pallas import tpu as pltpu
from jax.experimental.pallas import tpu_sc as plsc

WIN = 5
COUT = 200
CD = 50
CH = 12
WD = 300
CV = 128


LW = 128


def _sc_gather_rows(table, tail128, idx):
    T = idx.shape[0]
    NWORK = 32
    per_w = T // NWORK
    CHUNK = 128
    nch = per_w // CHUNK
    idx3d = idx.reshape(NWORK, nch, CHUNK)
    mesh = plsc.VectorSubcoreMesh(core_axis_name="c", subcore_axis_name="s")
    out_t = jax.ShapeDtypeStruct((T, LW), jnp.float32)

    @functools.partial(
        pl.kernel,
        mesh=mesh,
        out_type=(out_t, out_t, out_t),
        compiler_params=pltpu.CompilerParams(use_tc_tiling_on_sc=True),
        scratch_types=[
            pltpu.VMEM((nch, CHUNK), jnp.int32),
            pltpu.VMEM((2, 3, CHUNK, LW), jnp.float32),
            pltpu.SemaphoreType.DMA,
            pltpu.SemaphoreType.DMA,
        ],
    )
    def gk(table_hbm, tail_hbm, idx_hbm, o0, o1, o2, idx_v, rows_v, sem_g, sem_o):
        wid = lax.axis_index("s") * 2 + lax.axis_index("c")
        base = wid * per_w
        pltpu.sync_copy(idx_hbm.at[wid], idx_v)
        srcs = [table_hbm.at[:, pl.ds(0, LW)],
                table_hbm.at[:, pl.ds(LW, LW)],
                tail_hbm]
        dsts = [o0, o1, o2]
        outs = [None, None]
        for j in range(nch):
            b = j % 2
            if outs[b] is not None:
                for o in outs[b]:
                    o.wait()
                outs[b] = None
            gs = [pltpu.async_copy(srcs[k].at[idx_v.at[j]], rows_v.at[b, k], sem_g)
                  for k in range(3)]
            for g in gs:
                g.wait()
            outs[b] = [
                pltpu.async_copy(
                    rows_v.at[b, k],
                    dsts[k].at[pl.ds(base + j * CHUNK, CHUNK)], sem_o)
                for k in range(3)]
        for ob in outs:
            if ob is not None:
                for o in ob:
                    o.wait()

    return gk(table, tail128, idx3d)


def _char_word_block(w0_ref, w1_ref, w2_ref, ch_ref, tab_ref, wf_ref, b_ref, out_ref):
    NWB = ch_ref.shape[0]
    cb = ch_ref[...]
    tab = tab_ref[...]
    dn = (((1,), (0,)), ((), ()))
    emb = []
    for t in range(CH):
        oh = (cb[:, t : t + 1]
              == lax.broadcasted_iota(jnp.int32, (NWB, CV), 1)).astype(jnp.float32)
        emb.append(lax.dot_general(oh, tab, dn, preferred_element_type=jnp.float32))
    zero = jnp.zeros((NWB, CD), jnp.float32)
    wf = wf_ref[...]
    m = None
    for t in range(CH):
        parts = []
        for s in range(-2, 3):
            tt = t + s
            parts.append(emb[tt] if 0 <= tt < CH else zero)
        x = jnp.concatenate(parts, axis=1)
        y = lax.dot_general(x, wf, dn, preferred_element_type=jnp.float32)
        m = y if m is None else jnp.maximum(m, y)
    feat = m + b_ref[...]
    out_ref[...] = jnp.concatenate(
        [w0_ref[...], w1_ref[...], w2_ref[:, : WD - 2 * LW], feat], axis=1)


def _char_concat(w0, w1, w2, chars_flat, tab, wflat, bias2, row_off):
    n_tok = chars_flat.shape[0]
    NWB = 256
    grid = (n_tok // NWB,)
    off_blk = row_off // NWB
    wspec = pl.BlockSpec((NWB, LW), lambda i: (off_blk + i, 0))
    return pl.pallas_call(
        _char_word_block,
        grid=grid,
        in_specs=[
            wspec, wspec, wspec,
            pl.BlockSpec((NWB, CH), lambda i: (i, 0)),
            pl.BlockSpec((CV, CD), lambda i: (0, 0)),
            pl.BlockSpec((WIN * CD, COUT), lambda i: (0, 0)),
            pl.BlockSpec((1, COUT), lambda i: (0, 0)),
        ],
        out_specs=pl.BlockSpec((NWB, WD + COUT), lambda i: (i, 0)),
        out_shape=jax.ShapeDtypeStruct((n_tok, WD + COUT), jnp.float32),
    )(w0, w1, w2, chars_flat, tab, wflat, bias2)


def _onehot_matmul(iref, tref, vocab):
    oh = (iref[...]
          == lax.broadcasted_iota(jnp.int32, (iref.shape[0], vocab), 1)
          ).astype(jnp.float32)
    return lax.dot_general(oh, tref[...], (((1,), (0,)), ((), ())),
                           preferred_element_type=jnp.float32)


def _p_small_block(pos_i, ner_i, r1_i, r2_i, pos_t, ner_t, rel_t, o1, o2, o3, o4):
    o1[...] = _onehot_matmul(pos_i, pos_t, 50)
    o2[...] = _onehot_matmul(ner_i, ner_t, 20)
    o3[...] = _onehot_matmul(r1_i, rel_t, 40)
    o4[...] = _onehot_matmul(r2_i, rel_t, 40)


def _p_small(pPos, pNer, pQRel, pCRel, pos_table, ner_table, rel_table):
    n_tok = pPos.shape[0]
    NT = 512
    grid = (n_tok // NT,)
    idx_spec = pl.BlockSpec((NT, 1), lambda i: (i, 0))
    full = lambda shp: pl.BlockSpec(shp, lambda i: (0, 0))
    return pl.pallas_call(
        _p_small_block,
        grid=grid,
        in_specs=[idx_spec, idx_spec, idx_spec, idx_spec,
                  full(pos_table.shape), full(ner_table.shape), full(rel_table.shape)],
        out_specs=[pl.BlockSpec((NT, 12), lambda i: (i, 0)),
                   pl.BlockSpec((NT, 8), lambda i: (i, 0)),
                   pl.BlockSpec((NT, 10), lambda i: (i, 0)),
                   pl.BlockSpec((NT, 10), lambda i: (i, 0))],
        out_shape=[jax.ShapeDtypeStruct((n_tok, 12), jnp.float32),
                   jax.ShapeDtypeStruct((n_tok, 8), jnp.float32),
                   jax.ShapeDtypeStruct((n_tok, 10), jnp.float32),
                   jax.ShapeDtypeStruct((n_tok, 10), jnp.float32)],
    )(pPos, pNer, pQRel, pCRel, pos_table, ner_table, rel_table)


def _q_small_block(pos_i, pos_t, o1):
    o1[...] = _onehot_matmul(pos_i, pos_t, 50)


def _q_small(qPos, pos_table):
    n_tok = qPos.shape[0]
    NT = 512
    grid = (n_tok // NT,)
    return pl.pallas_call(
        _q_small_block,
        grid=grid,
        in_specs=[pl.BlockSpec((NT, 1), lambda i: (i, 0)),
                  pl.BlockSpec(pos_table.shape, lambda i: (0, 0))],
        out_specs=pl.BlockSpec((NT, 12), lambda i: (i, 0)),
        out_shape=jax.ShapeDtypeStruct((n_tok, 12), jnp.float32),
    )(qPos, pos_table)


def kernel(p, q, c, pPos, pNer, qPos, pQRel, pCRel, pChars, qChars, cChars,
           word_table, pos_table, ner_table, rel_table, char_table, conv_w, conv_b):
    B, PL = p.shape
    QL = q.shape[1]
    CL = c.shape[1]
    i32 = jnp.int32

    idx_all = jnp.concatenate(
        [p.reshape(-1), q.reshape(-1), c.reshape(-1)]).astype(i32)
    tail128 = jnp.pad(word_table[:, 2 * LW :], ((0, 0), (0, 3 * LW - WD)))
    w0, w1, w2 = _sc_gather_rows(word_table, tail128, idx_all)

    wflat = conv_w.transpose(2, 1, 0).reshape(WIN * CD, COUT)
    bias2 = conv_b.reshape(1, COUT)

    pEmb = _char_concat(w0, w1, w2, pChars.reshape(B * PL, CH).astype(i32),
                        char_table, wflat, bias2, 0).reshape(B, PL, WD + COUT)
    qEmb = _char_concat(w0, w1, w2, qChars.reshape(B * QL, CH).astype(i32),
                        char_table, wflat, bias2, B * PL).reshape(B, QL, WD + COUT)
    cEmb = _char_concat(w0, w1, w2, cChars.reshape(B * CL, CH).astype(i32),
                        char_table, wflat, bias2, B * (PL + QL)).reshape(B, CL, WD + COUT)

    o1, o2, o3, o4 = _p_small(pPos.reshape(B * PL, 1).astype(i32),
                              pNer.reshape(B * PL, 1).astype(i32),
                              pQRel.reshape(B * PL, 1).astype(i32),
                              pCRel.reshape(B * PL, 1).astype(i32),
                              pos_table, ner_table, rel_table)
    qPosEmb = _q_small(qPos.reshape(B * QL, 1).astype(i32),
                       pos_table).reshape(B, QL, 12)

    return (pEmb, qEmb, cEmb,
            o1.reshape(B, PL, 12), o2.reshape(B, PL, 8), qPosEmb,
            o3.reshape(B, PL, 10), o4.reshape(B, PL, 10))
